# baseline (speedup 1.0000x reference)
import functools

import jax
import jax.numpy as jnp
from jax import lax
from jax.experimental import pallas as pl
from jax.experimental.pallas import tpu as pltpu
from jax.experimental.pallas import tpu_sc as plsc


def _make_gather(V, D, BT, S):
    B = BT * S
    info = plsc.get_sparse_core_info()
    NC, NS = info.num_cores, info.num_subcores
    NW = NC * NS
    assert B % NW == 0
    b_per_w = B // NW
    assert S % b_per_w == 0
    w_per_row = S // b_per_w
    CHUNK = 64
    assert b_per_w % CHUNK == 0
    n_chunks = b_per_w // CHUNK
    assert n_chunks % 2 == 0 and n_chunks >= 4

    mesh = plsc.VectorSubcoreMesh(core_axis_name="c", subcore_axis_name="s")

    @functools.partial(
        pl.kernel,
        mesh=mesh,
        out_type=jax.ShapeDtypeStruct((B, D), jnp.float32),
        scratch_types=[
            pltpu.VMEM((b_per_w,), jnp.int32),
            pltpu.VMEM((CHUNK, D), jnp.float32),
            pltpu.VMEM((CHUNK, D), jnp.float32),
            pltpu.SemaphoreType.DMA,
            pltpu.SemaphoreType.DMA,
            pltpu.SemaphoreType.DMA,
            pltpu.SemaphoreType.DMA,
        ],
    )
    def k(tok_hbm, table_hbm, out_hbm, idx_v, rows0, rows1, g0, g1, w0, w1):
        wid = lax.axis_index("s") * NC + lax.axis_index("c")
        base = wid * b_per_w
        row = wid // w_per_row
        col = (wid % w_per_row) * b_per_w
        pltpu.sync_copy(tok_hbm.at[row, pl.ds(col, b_per_w)], idx_v)

        bufs = (rows0, rows1)
        gsems = (g0, g1)
        wsems = (w0, w1)

        def gather_desc(c, par):
            idx_slice = idx_v.at[pl.ds(c * CHUNK, CHUNK)]
            return pltpu.make_async_copy(
                table_hbm.at[idx_slice], bufs[par], gsems[par])

        def write_desc(c, par):
            dst = out_hbm.at[pl.ds(base + c * CHUNK, CHUNK)]
            return pltpu.make_async_copy(bufs[par], dst, wsems[par])

        def step(c, par, first, last):
            if not first:
                write_desc(c - 1, par ^ 1).wait()
            if not last:
                gather_desc(c + 1, par ^ 1).start()
            gather_desc(c, par).wait()
            write_desc(c, par).start()

        gather_desc(0, 0).start()
        step(0, 0, first=True, last=False)

        def body(i, carry):
            step(2 * i + 1, 1, first=False, last=False)
            step(2 * i + 2, 0, first=False, last=False)
            return carry

        lax.fori_loop(0, (n_chunks - 2) // 2, body, 0)

        step(n_chunks - 1, 1, first=False, last=True)
        write_desc(n_chunks - 1, 1).wait()

    return k


def kernel(tokens, W_E):
    B_, S_ = tokens.shape
    V, D = W_E.shape
    out = _make_gather(V, D, B_, S_)(tokens.astype(jnp.int32), W_E)
    return out.reshape(B_, S_, D)


# speedup vs baseline: 1.0073x; 1.0073x over previous
import functools

import jax
import jax.numpy as jnp
from jax import lax
from jax.experimental import pallas as pl
from jax.experimental.pallas import tpu as pltpu
from jax.experimental.pallas import tpu_sc as plsc


def _make_gather(V, D, BT, S):
    B = BT * S
    info = plsc.get_sparse_core_info()
    NC, NS = info.num_cores, info.num_subcores
    NW = NC * NS
    assert B % NW == 0
    b_per_w = B // NW
    assert S % b_per_w == 0
    w_per_row = S // b_per_w
    CHUNK = 32
    assert b_per_w % CHUNK == 0
    n_chunks = b_per_w // CHUNK
    assert (n_chunks - 4) % 4 == 0 and n_chunks >= 8

    mesh = plsc.VectorSubcoreMesh(core_axis_name="c", subcore_axis_name="s")

    NBUF = 4

    @functools.partial(
        pl.kernel,
        mesh=mesh,
        out_type=jax.ShapeDtypeStruct((B, D), jnp.float32),
        scratch_types=[
            pltpu.VMEM((b_per_w,), jnp.int32),
        ]
        + [pltpu.VMEM((CHUNK, D), jnp.float32)] * NBUF
        + [pltpu.SemaphoreType.DMA] * (2 * NBUF),
    )
    def k(tok_hbm, table_hbm, out_hbm, idx_v, *rest):
        bufs = rest[:NBUF]
        gsems = rest[NBUF:2 * NBUF]
        wsems = rest[2 * NBUF:]
        wid = lax.axis_index("s") * NC + lax.axis_index("c")
        base = wid * b_per_w
        row = wid // w_per_row
        col = (wid % w_per_row) * b_per_w
        pltpu.sync_copy(tok_hbm.at[row, pl.ds(col, b_per_w)], idx_v)

        def gather_desc(c, par):
            idx_slice = idx_v.at[pl.ds(c * CHUNK, CHUNK)]
            return pltpu.make_async_copy(
                table_hbm.at[idx_slice], bufs[par], gsems[par])

        def write_desc(c, par):
            dst = out_hbm.at[pl.ds(base + c * CHUNK, CHUNK)]
            return pltpu.make_async_copy(bufs[par], dst, wsems[par])

        def pos(c, par, do_issue, do_waitw):
            gather_desc(c, par).wait()
            write_desc(c, par).start()
            if do_issue:
                if do_waitw:
                    write_desc(c - 1, (par + 3) % NBUF).wait()
                gather_desc(c + 3, (par + 3) % NBUF).start()

        for c in range(NBUF - 1):
            gather_desc(c, c).start()
        pos(0, 0, True, False)

        def body(i, carry):
            for kk in range(NBUF):
                pos(NBUF * i + 1 + kk, (1 + kk) % NBUF, True, True)
            return carry

        lax.fori_loop(0, (n_chunks - NBUF) // NBUF, body, 0)

        for c in range(n_chunks - 3, n_chunks):
            pos(c, c % NBUF, False, False)
        for c in range(n_chunks - NBUF, n_chunks):
            write_desc(c, c % NBUF).wait()

    return k


def kernel(tokens, W_E):
    B_, S_ = tokens.shape
    V, D = W_E.shape
    out = _make_gather(V, D, B_, S_)(tokens.astype(jnp.int32), W_E)
    return out.reshape(B_, S_, D)
